# baseline (device time: 257234 ns/iter reference)
import jax
import jax.numpy as jnp
from jax import lax
from jax.experimental import pallas as pl
from jax.experimental.pallas import tpu as pltpu

NY = 4
NZ = 4
NX = 2
NLINE = 4
NSTEP = NLINE - 1
NQ = 4


def kernel(x, dy):
    m, d = x.shape
    _, f = dy.shape
    fb = f // (NY * NZ)
    qw = fb // NQ
    dh = d // NX

    def body(x_ref, dy_ref, out_ref,
             xh_ref, xh2_ref, dys_ref, csend_ref, xrecv_ref,
             dy_copy_sems, xh_copy_sem, xh2_copy_sem, xsend_sems, xrecv_sems,
             ag_send_sems, ag_recv_sems):
        gx = lax.axis_index("x")
        gy = lax.axis_index("y")
        gz = lax.axis_index("z")
        peer = 1 - gx
        b = gy * NZ + gz

        def nbr_y(delta):
            return (gx, gy + delta, gz)

        def nbr_z(delta):
            return (gx, gy, gz + delta)

        def cfg(q, p):
            zfirst = (q % 2 == 0)
            on_z = (p == 0) == zfirst
            sem_base = (q * 2 + p) * 2 * NSTEP
            if p == 0:
                if zfirst:
                    sl = lambda k: [((gy * NZ + k) * fb + q * qw, qw)]
                else:
                    sl = lambda k: [((k * NZ + gz) * fb + q * qw, qw)]
            else:
                if zfirst:
                    sl = lambda k: [((k * NZ + t) * fb + q * qw, qw)
                                    for t in range(NZ)]
                else:
                    sl = lambda k: [((t * NZ + k) * fb + q * qw, qw)
                                    for t in range(NY)]
            pos = gz if on_z else gy
            nbr = nbr_z if on_z else nbr_y
            return pos, sl, nbr, sem_base

        def descs(q, p, k, delta, sem_idx):
            _, sl, nbr, _ = cfg(q, p)
            return [
                pltpu.make_async_remote_copy(
                    src_ref=out_ref.at[:, pl.ds(off, w)],
                    dst_ref=out_ref.at[:, pl.ds(off, w)],
                    send_sem=ag_send_sems.at[sem_idx],
                    recv_sem=ag_recv_sems.at[sem_idx],
                    device_id=nbr(delta),
                    device_id_type=pl.DeviceIdType.MESH,
                )
                for off, w in sl(k)
            ]

        def step_ios(q, p, s):
            pos, _, _, base = cfg(q, p)
            return [
                dict(sem=base + s,
                     send_cond=(pos < NLINE - 1) & (pos - s >= 0),
                     send_k=pos - s,
                     recv_cond=(pos - 1 - s >= 0),
                     recv_k=pos - 1 - s,
                     delta=1),
                dict(sem=base + NSTEP + s,
                     send_cond=(pos > 0) & (pos + s <= NLINE - 1),
                     send_k=pos + s,
                     recv_cond=(pos + 1 + s <= NLINE - 1),
                     recv_k=pos + 1 + s,
                     delta=-1),
            ]

        def ag_send(q, p, s):
            for io in step_ios(q, p, s):
                @pl.when(io["send_cond"])
                def _(io=io):
                    for rd in descs(q, p, io["send_k"], io["delta"],
                                    io["sem"]):
                        rd.start()

        def ag_recv_wait(q, p, s):
            for io in step_ios(q, p, s):
                @pl.when(io["recv_cond"])
                def _(io=io):
                    for rd in descs(q, p, io["recv_k"], -io["delta"],
                                    io["sem"]):
                        rd.wait_recv()

        def ag_send_wait(q, p, s):
            for io in step_ios(q, p, s):
                @pl.when(io["send_cond"])
                def _(io=io):
                    for rd in descs(q, p, io["send_k"], io["delta"],
                                    io["sem"]):
                        rd.wait_send()

        dy_cps = [
            pltpu.make_async_copy(
                dy_ref.at[:, pl.ds(b * fb + q * qw, qw)],
                dys_ref.at[q % 2],
                dy_copy_sems.at[q % 2],
            )
            for q in range(NQ)
        ]
        dy_cps[0].start()
        xh_cp = pltpu.make_async_copy(
            x_ref.at[:, pl.ds(peer * dh, dh)], xh_ref, xh_copy_sem
        )
        xh_cp.start()
        xh2_cp = pltpu.make_async_copy(
            x_ref.at[:, pl.ds(gx * dh, dh)], xh2_ref, xh2_copy_sem
        )
        xh2_cp.start()
        xh_cp.wait()

        dims = (((0,), (0,)), ((), ()))

        def xchg_desc(q):
            return pltpu.make_async_remote_copy(
                src_ref=csend_ref.at[:, pl.ds(q * qw, qw)],
                dst_ref=xrecv_ref.at[:, pl.ds(q * qw, qw)],
                send_sem=xsend_sems.at[q],
                recv_sem=xrecv_sems.at[q],
                device_id=(peer, gy, gz),
                device_id_type=pl.DeviceIdType.MESH,
            )

        def peer_dot(q):
            csend_ref[:, q * qw:(q + 1) * qw] = lax.dot_general(
                xh_ref[:, :], dys_ref[q % 2], dims,
                preferred_element_type=jnp.float32,
            )
            xchg_desc(q).start()

        def mine_dot(q):
            c_mine = lax.dot_general(
                xh2_ref[:, :], dys_ref[q % 2], dims,
                preferred_element_type=jnp.float32,
            )
            xchg_desc(q).wait()
            out_ref[:, pl.ds(b * fb + q * qw, qw)] = \
                c_mine + xrecv_ref[:, q * qw:(q + 1) * qw]
            ag_send(q, 0, 0)

        dy_cps[0].wait()
        dy_cps[1].start()
        peer_dot(0)
        xh2_cp.wait()
        mine_dot(0)
        dy_cps[2].start()
        dy_cps[1].wait()
        peer_dot(1)
        mine_dot(1)
        dy_cps[3].start()
        dy_cps[2].wait()
        peer_dot(2)
        dy_cps[3].wait()
        peer_dot(3)
        mine_dot(2)
        mine_dot(3)

        for s in range(NSTEP):
            for q in range(NQ):
                ag_recv_wait(q, 0, s)
                ag_send_wait(q, 0, s)
                if s + 1 < NSTEP:
                    ag_send(q, 0, s + 1)
                else:
                    ag_send(q, 1, 0)
        for s in range(NSTEP):
            for q in range(NQ):
                ag_recv_wait(q, 1, s)
                ag_send_wait(q, 1, s)
                if s + 1 < NSTEP:
                    ag_send(q, 1, s + 1)

    return pl.pallas_call(
        body,
        out_shape=jax.ShapeDtypeStruct((dh, f), jnp.float32),
        in_specs=[
            pl.BlockSpec(memory_space=pl.ANY),
            pl.BlockSpec(memory_space=pl.ANY),
        ],
        out_specs=pl.BlockSpec(memory_space=pltpu.MemorySpace.VMEM),
        scratch_shapes=[
            pltpu.VMEM((m, dh), jnp.float32),
            pltpu.VMEM((m, dh), jnp.float32),
            pltpu.VMEM((2, m, qw), jnp.float32),
            pltpu.VMEM((dh, fb), jnp.float32),
            pltpu.VMEM((dh, fb), jnp.float32),
            pltpu.SemaphoreType.DMA((2,)),
            pltpu.SemaphoreType.DMA,
            pltpu.SemaphoreType.DMA,
            pltpu.SemaphoreType.DMA((NQ,)),
            pltpu.SemaphoreType.DMA((NQ,)),
            pltpu.SemaphoreType.DMA((NQ * 2 * 2 * NSTEP,)),
            pltpu.SemaphoreType.DMA((NQ * 2 * 2 * NSTEP,)),
        ],
        compiler_params=pltpu.CompilerParams(
            vmem_limit_bytes=63 * 1024 * 1024,
        ),
    )(x, dy)


# device time: 244605 ns/iter; 1.0516x vs baseline; 1.0516x over previous
import jax
import jax.numpy as jnp
from jax import lax
from jax.experimental import pallas as pl
from jax.experimental.pallas import tpu as pltpu

NY = 4
NZ = 4
NX = 2
NLINE = 4
NSTEP = NLINE - 1
NQ = 4


def kernel(x, dy):
    m, d = x.shape
    _, f = dy.shape
    fb = f // (NY * NZ)
    qw = fb // NQ
    dh = d // NX

    def body(x_ref, dy_ref, out_ref,
             xh_ref, dys_ref, csend_ref, xrecv_ref,
             dy_copy_sem, xh_copy_sem, xsend_sems, xrecv_sems,
             ag_send_sems, ag_recv_sems):
        gx = lax.axis_index("x")
        gy = lax.axis_index("y")
        gz = lax.axis_index("z")
        peer = 1 - gx
        b = gy * NZ + gz

        def nbr_y(delta):
            return (gx, gy + delta, gz)

        def nbr_z(delta):
            return (gx, gy, gz + delta)

        def cfg(q, p):
            zfirst = (q % 2 == 0)
            on_z = (p == 0) == zfirst
            sem_base = (q * 2 + p) * 2 * NSTEP
            if p == 0:
                if zfirst:
                    sl = lambda k: [((gy * NZ + k) * fb + q * qw, qw)]
                else:
                    sl = lambda k: [((k * NZ + gz) * fb + q * qw, qw)]
            else:
                if zfirst:
                    sl = lambda k: [((k * NZ + t) * fb + q * qw, qw)
                                    for t in range(NZ)]
                else:
                    sl = lambda k: [((t * NZ + k) * fb + q * qw, qw)
                                    for t in range(NY)]
            pos = gz if on_z else gy
            nbr = nbr_z if on_z else nbr_y
            return pos, sl, nbr, sem_base

        def descs(q, p, k, delta, sem_idx):
            _, sl, nbr, _ = cfg(q, p)
            return [
                pltpu.make_async_remote_copy(
                    src_ref=out_ref.at[:, pl.ds(off, w)],
                    dst_ref=out_ref.at[:, pl.ds(off, w)],
                    send_sem=ag_send_sems.at[sem_idx],
                    recv_sem=ag_recv_sems.at[sem_idx],
                    device_id=nbr(delta),
                    device_id_type=pl.DeviceIdType.MESH,
                )
                for off, w in sl(k)
            ]

        def step_ios(q, p, s):
            pos, _, _, base = cfg(q, p)
            return [
                dict(sem=base + s,
                     send_cond=(pos < NLINE - 1) & (pos - s >= 0),
                     send_k=pos - s,
                     recv_cond=(pos - 1 - s >= 0),
                     recv_k=pos - 1 - s,
                     delta=1),
                dict(sem=base + NSTEP + s,
                     send_cond=(pos > 0) & (pos + s <= NLINE - 1),
                     send_k=pos + s,
                     recv_cond=(pos + 1 + s <= NLINE - 1),
                     recv_k=pos + 1 + s,
                     delta=-1),
            ]

        def ag_send(q, p, s):
            for io in step_ios(q, p, s):
                @pl.when(io["send_cond"])
                def _(io=io):
                    for rd in descs(q, p, io["send_k"], io["delta"],
                                    io["sem"]):
                        rd.start()

        def ag_recv_wait(q, p, s):
            for io in step_ios(q, p, s):
                @pl.when(io["recv_cond"])
                def _(io=io):
                    for rd in descs(q, p, io["recv_k"], -io["delta"],
                                    io["sem"]):
                        rd.wait_recv()

        def ag_send_wait(q, p, s):
            for io in step_ios(q, p, s):
                @pl.when(io["send_cond"])
                def _(io=io):
                    for rd in descs(q, p, io["send_k"], io["delta"],
                                    io["sem"]):
                        rd.wait_send()

        dy_cp = pltpu.make_async_copy(
            dy_ref.at[:, pl.ds(b * fb, fb)], dys_ref, dy_copy_sem
        )
        dy_cp.start()
        xh_cp = pltpu.make_async_copy(
            x_ref.at[:, pl.ds(peer * dh, dh)], xh_ref, xh_copy_sem
        )
        xh_cp.start()
        dy_cp.wait()
        xh_cp.wait()

        dims = (((0,), (0,)), ((), ()))

        def xchg_desc(q):
            return pltpu.make_async_remote_copy(
                src_ref=csend_ref.at[:, pl.ds(q * qw, qw)],
                dst_ref=xrecv_ref.at[:, pl.ds(q * qw, qw)],
                send_sem=xsend_sems.at[q],
                recv_sem=xrecv_sems.at[q],
                device_id=(peer, gy, gz),
                device_id_type=pl.DeviceIdType.MESH,
            )

        for q in range(NQ):
            csend_ref[:, q * qw:(q + 1) * qw] = lax.dot_general(
                xh_ref[:, :], dys_ref[:, q * qw:(q + 1) * qw], dims,
                preferred_element_type=jnp.float32,
            )
            xchg_desc(q).start()

        xh2_cp = pltpu.make_async_copy(
            x_ref.at[:, pl.ds(gx * dh, dh)], xh_ref, xh_copy_sem
        )
        xh2_cp.start()
        xh2_cp.wait()

        for q in range(NQ):
            c_mine = lax.dot_general(
                xh_ref[:, :], dys_ref[:, q * qw:(q + 1) * qw], dims,
                preferred_element_type=jnp.float32,
            )
            xchg_desc(q).wait()
            out_ref[:, pl.ds(b * fb + q * qw, qw)] = \
                c_mine + xrecv_ref[:, q * qw:(q + 1) * qw]
            ag_send(q, 0, 0)

        for s in range(NSTEP):
            for q in range(NQ):
                ag_recv_wait(q, 0, s)
                ag_send_wait(q, 0, s)
                if s + 1 < NSTEP:
                    ag_send(q, 0, s + 1)
                else:
                    ag_send(q, 1, 0)
        for s in range(NSTEP):
            for q in range(NQ):
                ag_recv_wait(q, 1, s)
                ag_send_wait(q, 1, s)
                if s + 1 < NSTEP:
                    ag_send(q, 1, s + 1)

    return pl.pallas_call(
        body,
        out_shape=jax.ShapeDtypeStruct((dh, f), jnp.float32),
        in_specs=[
            pl.BlockSpec(memory_space=pl.ANY),
            pl.BlockSpec(memory_space=pl.ANY),
        ],
        out_specs=pl.BlockSpec(memory_space=pltpu.MemorySpace.VMEM),
        scratch_shapes=[
            pltpu.VMEM((m, dh), jnp.float32),
            pltpu.VMEM((m, fb), jnp.float32),
            pltpu.VMEM((dh, fb), jnp.float32),
            pltpu.VMEM((dh, fb), jnp.float32),
            pltpu.SemaphoreType.DMA,
            pltpu.SemaphoreType.DMA,
            pltpu.SemaphoreType.DMA((NQ,)),
            pltpu.SemaphoreType.DMA((NQ,)),
            pltpu.SemaphoreType.DMA((NQ * 2 * 2 * NSTEP,)),
            pltpu.SemaphoreType.DMA((NQ * 2 * 2 * NSTEP,)),
        ],
        compiler_params=pltpu.CompilerParams(
            vmem_limit_bytes=60 * 1024 * 1024,
        ),
    )(x, dy)
